# baseline (speedup 1.0000x reference)
import functools

import jax
import jax.numpy as jnp
from jax import lax
from jax.experimental import pallas as pl
from jax.experimental.pallas import tpu as pltpu
from jax.experimental.pallas import tpu_sc as plsc

_B = 32
_N = 12288
_NK = 8
_C = _NK + 1
_K = 10
_RK = 0.1
_CHUNK = 1024
_NCH = _N // _CHUNK
_GROUPS = _CHUNK // 16

_mesh = plsc.VectorSubcoreMesh(core_axis_name="c", subcore_axis_name="s")


@functools.partial(
    pl.kernel,
    out_type=jax.ShapeDtypeStruct((_B, 32), jnp.float32),
    mesh=_mesh,
    compiler_params=pltpu.CompilerParams(needs_layout_passes=False),
    scratch_types=[
        pltpu.VMEM((_CHUNK * 24,), jnp.float32),
        pltpu.VMEM((_CHUNK * 24,), jnp.float32),
        pltpu.VMEM((_CHUNK * 3,), jnp.float32),
        pltpu.VMEM((_CHUNK * 3,), jnp.float32),
        pltpu.VMEM((_CHUNK * 2,), jnp.float32),
        pltpu.VMEM((_CHUNK * 2,), jnp.float32),
        pltpu.VMEM((_C * 6 * 16,), jnp.float32),
        pltpu.VMEM((32,), jnp.float32),
        pltpu.SemaphoreType.DMA,
        pltpu.SemaphoreType.DMA,
        pltpu.SemaphoreType.DMA,
    ],
)
def _sc_vote(kpts_hbm, cpt_hbm, seg_hbm, pcld_hbm, out_hbm,
             kbuf0, kbuf1, cbuf0, cbuf1, sbuf0, sbuf1, vbuf, obuf,
             sem0, sem1, rsem):
    b = lax.axis_index("s") * 2 + lax.axis_index("c")
    base = b * _N
    iota = lax.iota(jnp.int32, 16)
    inf16 = jnp.full((16,), jnp.inf, jnp.float32)
    zero16 = jnp.zeros((16,), jnp.int32)
    kbufs = (kbuf0, kbuf1)
    cbufs = (cbuf0, cbuf1)
    sbufs = (sbuf0, sbuf1)
    sems = (sem0, sem1)

    def start_chunk(ch, slot):
        off = base + ch * _CHUNK
        return (
            pltpu.async_copy(kpts_hbm.at[pl.ds(off * 24, _CHUNK * 24)],
                             kbufs[slot], sems[slot]),
            pltpu.async_copy(cpt_hbm.at[pl.ds(off * 3, _CHUNK * 3)],
                             cbufs[slot], sems[slot]),
            pltpu.async_copy(seg_hbm.at[pl.ds(off * 2, _CHUNK * 2)],
                             sbufs[slot], sems[slot]),
        )

    def process_chunk(ch, slot, acc):
        kb = kbufs[slot]
        cb = cbufs[slot]
        sb = sbufs[slot]

        def group(g, acc):
            rows = g * 16 + iota
            rows2 = rows * 2
            rows3 = rows * 3
            rows24 = rows * 24
            s0 = plsc.load_gather(sb, [rows2])
            s1 = plsc.load_gather(sb, [rows2 + 1])
            pen = jnp.where(s0 >= s1, jnp.float32(1e10), jnp.float32(0.0))
            pid = ch * _CHUNK + g * 16 + iota
            out = []
            for c in range(_C):
                if c < _NK:
                    x = plsc.load_gather(kb, [rows24 + (3 * c)])
                    y = plsc.load_gather(kb, [rows24 + (3 * c + 1)])
                    z = plsc.load_gather(kb, [rows24 + (3 * c + 2)])
                else:
                    x = plsc.load_gather(cb, [rows3])
                    y = plsc.load_gather(cb, [rows3 + 1])
                    z = plsc.load_gather(cb, [rows3 + 2])
                s = ((x * x + y * y) + z * z) + pen
                sk, sv = plsc.sort_key_val(s, pid)
                rk = lax.rev(sk, (0,))
                rv = lax.rev(sv, (0,))
                ak, av = acc[2 * c], acc[2 * c + 1]
                take = rk < ak
                mk = jnp.where(take, rk, ak)
                mv = jnp.where(take, rv, av)
                ak2, av2 = plsc.sort_key_val(mk, mv)
                out += [ak2, av2]
            return tuple(out)

        return lax.fori_loop(0, _GROUPS, group, acc)

    acc = tuple(inf16 if i % 2 == 0 else zero16 for i in range(2 * _C))
    pending = {0: start_chunk(0, 0)}
    for ch in range(_NCH):
        slot = ch & 1
        if ch + 1 < _NCH:
            pending[ch + 1] = start_chunk(ch + 1, (ch + 1) & 1)
        for h in pending.pop(ch):
            h.wait()
        acc = process_chunk(ch, slot, acc)

    handles = []
    for c in range(_C):
        gidx = acc[2 * c + 1] + base
        g24 = gidx * 24
        g3 = gidx * 3
        for d in range(3):
            odst = vbuf.at[pl.ds((c * 6 + d) * 16, 16)]
            pdst = vbuf.at[pl.ds((c * 6 + 3 + d) * 16, 16)]
            if c < _NK:
                handles.append(pltpu.async_copy(
                    kpts_hbm.at[g24 + (3 * c + d)], odst, rsem))
            else:
                handles.append(pltpu.async_copy(
                    cpt_hbm.at[g3 + d], odst, rsem))
            handles.append(pltpu.async_copy(pcld_hbm.at[g3 + d], pdst, rsem))
    for h in handles:
        h.wait()

    valid = iota < _K
    vec0 = jnp.zeros((16,), jnp.float32)
    vec1 = jnp.zeros((16,), jnp.float32)
    for c in range(_C):
        for d in range(3):
            v = (vbuf[pl.ds((c * 6 + d) * 16, 16)]
                 + vbuf[pl.ds((c * 6 + 3 + d) * 16, 16)])
            mean = jnp.sum(jnp.where(valid, v, jnp.float32(0.0))) * _RK
            dev = v - mean
            sq = dev * dev
            var = jnp.sum(jnp.where(valid, sq, jnp.float32(0.0))) * _RK
            m = valid & (sq <= var)
            cnt = jnp.sum(jnp.where(m, jnp.float32(1.0), jnp.float32(0.0)))
            vs = jnp.sum(jnp.where(m, v, jnp.float32(0.0)))
            voted = (jnp.broadcast_to(vs, (16,))
                     / jnp.broadcast_to(cnt + jnp.float32(1e-8), (16,)))
            j = 3 * c + d
            if j < 16:
                vec0 = vec0 + jnp.where(iota == j, voted, jnp.float32(0.0))
            else:
                vec1 = vec1 + jnp.where(iota == (j - 16), voted,
                                        jnp.float32(0.0))
    obuf[pl.ds(0, 16)] = vec0
    obuf[pl.ds(16, 16)] = vec1
    pltpu.sync_copy(obuf, out_hbm.at[b])


def _kabsch_body(a_ref, b_ref, r_ref, t_ref):
    Ad = [a_ref[d] for d in range(3)]
    Bd = [b_ref[d] for d in range(3)]
    cA = [jnp.sum(Ad[d], axis=1, keepdims=True) / 9.0 for d in range(3)]
    cB = [jnp.sum(Bd[d], axis=1, keepdims=True) / 9.0 for d in range(3)]
    Am = [Ad[d] - cA[d] for d in range(3)]
    Bm = [Bd[d] - cB[d] for d in range(3)]
    H = [[jnp.sum(Am[d] * Bm[e], axis=1, keepdims=True) for e in range(3)]
         for d in range(3)]
    (sxx, sxy, sxz), (syx, syy, syz), (szx, szy, szz) = H
    K4 = [
        [sxx + syy + szz, syz - szy, szx - sxz, sxy - syx],
        [syz - szy, sxx - syy - szz, sxy + syx, szx + sxz],
        [szx - sxz, sxy + syx, -sxx + syy - szz, syz + szy],
        [sxy - syx, szx + sxz, syz + szy, -sxx - syy + szz],
    ]
    one = jnp.ones_like(sxx)
    nil = jnp.zeros_like(sxx)
    Q = [[one if i == j else nil for j in range(4)] for i in range(4)]
    for _ in range(6):
        for p_, q_ in ((0, 1), (0, 2), (0, 3), (1, 2), (1, 3), (2, 3)):
            apq = K4[p_][q_]
            safe = jnp.abs(apq) > 1e-30
            theta = (K4[q_][q_] - K4[p_][p_]) / jnp.where(safe, 2.0 * apq, 1.0)
            sgn = jnp.where(theta >= 0, 1.0, -1.0)
            t = sgn / (jnp.abs(theta) + jnp.sqrt(theta * theta + 1.0))
            t = jnp.where(safe, t, 0.0)
            c = lax.rsqrt(1.0 + t * t)
            s = t * c
            for i in range(4):
                kip, kiq = K4[i][p_], K4[i][q_]
                K4[i][p_] = c * kip - s * kiq
                K4[i][q_] = s * kip + c * kiq
            for j in range(4):
                kpj, kqj = K4[p_][j], K4[q_][j]
                K4[p_][j] = c * kpj - s * kqj
                K4[q_][j] = s * kpj + c * kqj
            for i in range(4):
                qip, qiq = Q[i][p_], Q[i][q_]
                Q[i][p_] = c * qip - s * qiq
                Q[i][q_] = s * qip + c * qiq
    w = [K4[i][i] for i in range(4)]
    q = [Q[i][0] for i in range(4)]
    bw = w[0]
    for j in range(1, 4):
        better = w[j] > bw
        bw = jnp.where(better, w[j], bw)
        q = [jnp.where(better, Q[i][j], q[i]) for i in range(4)]
    q0, q1, q2, q3 = q
    r = [
        [1 - 2 * (q2 * q2 + q3 * q3), 2 * (q1 * q2 - q0 * q3), 2 * (q1 * q3 + q0 * q2)],
        [2 * (q1 * q2 + q0 * q3), 1 - 2 * (q1 * q1 + q3 * q3), 2 * (q2 * q3 - q0 * q1)],
        [2 * (q1 * q3 - q0 * q2), 2 * (q2 * q3 + q0 * q1), 1 - 2 * (q1 * q1 + q2 * q2)],
    ]
    for d in range(3):
        td = cB[d]
        for e in range(3):
            r_ref[d, e] = r[d][e][:, 0]
            td = td - r[d][e] * cA[e]
        t_ref[d] = td[:, 0]


def kernel(pcld_input, kpts_pre_input, cpt_pre_input, seg_pre_input,
           mesh_kpts_input):
    kpts_flat = kpts_pre_input.reshape(_B * _N * _NK * 3)
    cpt_flat = cpt_pre_input.reshape(_B * _N * 3)
    seg_flat = seg_pre_input.reshape(_B * _N * 2)
    pcld_flat = pcld_input.reshape(_B * _N * 3)
    sc_out = _sc_vote(kpts_flat, cpt_flat, seg_flat, pcld_flat)
    voted = sc_out[:, :27].reshape(_B, _C, 3)
    a_t = jnp.transpose(mesh_kpts_input, (2, 0, 1))
    b_t = jnp.transpose(voted, (2, 0, 1))
    r3, t3 = pl.pallas_call(
        _kabsch_body,
        out_shape=(
            jax.ShapeDtypeStruct((3, 3, _B), jnp.float32),
            jax.ShapeDtypeStruct((3, _B), jnp.float32),
        ),
    )(a_t, b_t)
    batch_r = jnp.transpose(r3, (2, 0, 1))
    batch_t = jnp.transpose(t3, (1, 0))
    return (batch_r, batch_t, voted)
